# baseline (speedup 1.0000x reference)
import functools

import jax
import jax.numpy as jnp
from jax import lax
from jax.experimental import pallas as pl
from jax.experimental.pallas import tpu as pltpu
from jax.experimental.pallas import tpu_sc as plsc

B = 16384
N_SPARSE = 26
N_DENSE = 13
VOCAB = 100000
LANES = 16
K_GROUPS = 4

_info = plsc.get_sparse_core_info()
NC, NS = _info.num_cores, _info.num_subcores
NW = NC * NS
RPW = B // NW
CHUNKS = RPW // LANES


def _sc_group_body(n_fields, n_extra, *refs):
    idx_hbm = refs[0]
    planes = refs[1:1 + n_fields]
    extras = refs[1 + n_fields:1 + n_fields + n_extra]
    out_hbm = refs[1 + n_fields + n_extra]
    idx_v, gat_v, ext_v, out_v, sem = refs[2 + n_fields + n_extra:]

    wid = lax.axis_index("s") * NC + lax.axis_index("c")
    base = wid * RPW
    seg = n_fields * RPW

    pltpu.sync_copy(idx_hbm.at[pl.ds(wid * seg, seg)], idx_v)
    copies = [
        pltpu.async_copy(
            planes[j].at[idx_v.at[pl.ds(j * RPW, RPW)]],
            gat_v.at[pl.ds(j * RPW, RPW)], sem)
        for j in range(n_fields)
    ]
    for e in range(n_extra):
        pltpu.sync_copy(extras[e].at[pl.ds(base, RPW)],
                        ext_v.at[pl.ds(e * RPW, RPW)])
    for c in copies:
        c.wait()

    def reduce(c, carry):
        o = c * LANES
        acc = gat_v[pl.ds(o, LANES)]
        for j in range(1, n_fields):
            acc = acc + gat_v[pl.ds(j * RPW + o, LANES)]
        for e in range(n_extra):
            acc = acc + ext_v[pl.ds(e * RPW + o, LANES)]
        out_v[pl.ds(o, LANES)] = acc
        return carry

    lax.fori_loop(0, CHUNKS, reduce, 0)
    pltpu.sync_copy(out_v, out_hbm.at[pl.ds(base, RPW)])


def _make_sc_call(n_fields, n_extra):
    mesh = plsc.VectorSubcoreMesh(core_axis_name="c", subcore_axis_name="s")
    return pl.kernel(
        functools.partial(_sc_group_body, n_fields, n_extra),
        mesh=mesh,
        out_type=jax.ShapeDtypeStruct((B,), jnp.float32),
        scratch_types=[
            pltpu.VMEM((n_fields * RPW,), jnp.int32),
            pltpu.VMEM((n_fields * RPW,), jnp.float32),
            pltpu.VMEM((max(n_extra, 1) * RPW,), jnp.float32),
            pltpu.VMEM((RPW,), jnp.float32),
            pltpu.SemaphoreType.DMA,
        ],
        compiler_params=pltpu.CompilerParams(needs_layout_passes=False),
    )


def _dense_body(xdt_ref, w_ref, out_ref):
    out_ref[:] = jnp.sum(xdt_ref[:] * w_ref[:], axis=0)


def _dense_matvec(X_dense, weight):
    return pl.pallas_call(
        _dense_body,
        out_shape=jax.ShapeDtypeStruct((B,), jnp.float32),
        in_specs=[
            pl.BlockSpec((N_DENSE, B), lambda: (0, 0)),
            pl.BlockSpec((N_DENSE, 1), lambda: (0, 0)),
        ],
        out_specs=pl.BlockSpec((B,), lambda: (0,)),
    )(X_dense.T, weight)


@jax.jit
def kernel(X_sparse, X_dense, tables, weight):
    gsz = [N_SPARSE // K_GROUPS + (1 if g < N_SPARSE % K_GROUPS else 0)
           for g in range(K_GROUPS)]
    starts = [sum(gsz[:g]) for g in range(K_GROUPS)]

    Xw = X_sparse.reshape(NW, RPW, N_SPARSE)
    dense = _dense_matvec(X_dense, weight)

    partials = []
    anchor = None
    for g in range(K_GROUPS):
        f0, nf = starts[g], gsz[g]
        raw = [tables[f, :, 0] for f in range(f0, f0 + nf)]
        if anchor is None:
            planes_g = lax.optimization_barrier(tuple(raw))
        else:
            barr = lax.optimization_barrier(tuple(raw) + (anchor,))
            planes_g = barr[:nf]
        anchor = planes_g[0]
        idx_g = Xw[:, :, f0:f0 + nf].transpose(0, 2, 1).reshape(-1)
        extras = [] if g < K_GROUPS - 1 else [dense] + partials
        run = _make_sc_call(nf, len(extras))
        out = run(idx_g, *planes_g, *extras)
        if g < K_GROUPS - 1:
            partials.append(out)
        else:
            final = out
    return final.reshape(B, 1)


# speedup vs baseline: 1.2616x; 1.2616x over previous
import functools

import jax
import jax.numpy as jnp
from jax import lax
from jax.experimental import pallas as pl
from jax.experimental.pallas import tpu as pltpu
from jax.experimental.pallas import tpu_sc as plsc

B = 16384
N_SPARSE = 26
N_DENSE = 13
VOCAB = 100000
LANES = 16
K_GROUPS = 2

_info = plsc.get_sparse_core_info()
NC, NS = _info.num_cores, _info.num_subcores
NW = NC * NS
RPW = B // NW
CHUNKS = RPW // LANES


def _sc_group_body(n_fields, n_extra, *refs):
    idx_hbm = refs[0]
    planes = refs[1:1 + n_fields]
    extras = refs[1 + n_fields:1 + n_fields + n_extra]
    out_hbm = refs[1 + n_fields + n_extra]
    idx_v, gat_v, ext_v, out_v, sem = refs[2 + n_fields + n_extra:]

    wid = lax.axis_index("s") * NC + lax.axis_index("c")
    base = wid * RPW
    seg = n_fields * RPW

    pltpu.sync_copy(idx_hbm.at[pl.ds(wid * seg, seg)], idx_v)
    copies = [
        pltpu.async_copy(
            planes[j].at[idx_v.at[pl.ds(j * RPW, RPW)]],
            gat_v.at[pl.ds(j * RPW, RPW)], sem)
        for j in range(n_fields)
    ]
    for e in range(n_extra):
        pltpu.sync_copy(extras[e].at[pl.ds(base, RPW)],
                        ext_v.at[pl.ds(e * RPW, RPW)])
    for c in copies:
        c.wait()

    def reduce(c, carry):
        o = c * LANES
        acc = gat_v[pl.ds(o, LANES)]
        for j in range(1, n_fields):
            acc = acc + gat_v[pl.ds(j * RPW + o, LANES)]
        for e in range(n_extra):
            acc = acc + ext_v[pl.ds(e * RPW + o, LANES)]
        out_v[pl.ds(o, LANES)] = acc
        return carry

    lax.fori_loop(0, CHUNKS, reduce, 0)
    pltpu.sync_copy(out_v, out_hbm.at[pl.ds(base, RPW)])


def _make_sc_call(n_fields, n_extra):
    mesh = plsc.VectorSubcoreMesh(core_axis_name="c", subcore_axis_name="s")
    return pl.kernel(
        functools.partial(_sc_group_body, n_fields, n_extra),
        mesh=mesh,
        out_type=jax.ShapeDtypeStruct((B,), jnp.float32),
        scratch_types=[
            pltpu.VMEM((n_fields * RPW,), jnp.int32),
            pltpu.VMEM((n_fields * RPW,), jnp.float32),
            pltpu.VMEM((max(n_extra, 1) * RPW,), jnp.float32),
            pltpu.VMEM((RPW,), jnp.float32),
            pltpu.SemaphoreType.DMA,
        ],
        compiler_params=pltpu.CompilerParams(needs_layout_passes=False),
    )


def _dense_body(xdt_ref, w_ref, out_ref):
    out_ref[:] = jnp.sum(xdt_ref[:] * w_ref[:], axis=0)


def _dense_matvec(X_dense, weight):
    return pl.pallas_call(
        _dense_body,
        out_shape=jax.ShapeDtypeStruct((B,), jnp.float32),
        in_specs=[
            pl.BlockSpec((N_DENSE, B), lambda: (0, 0)),
            pl.BlockSpec((N_DENSE, 1), lambda: (0, 0)),
        ],
        out_specs=pl.BlockSpec((B,), lambda: (0,)),
    )(X_dense.T, weight)


@jax.jit
def kernel(X_sparse, X_dense, tables, weight):
    gsz = [16, 10]
    starts = [0, 16]

    Xw = X_sparse.reshape(NW, RPW, N_SPARSE)
    dense = _dense_matvec(X_dense, weight)

    partials = []
    for g in range(K_GROUPS):
        f0, nf = starts[g], gsz[g]
        planes_g = [tables[f, :, 0] for f in range(f0, f0 + nf)]
        idx_g = Xw[:, :, f0:f0 + nf].transpose(0, 2, 1).reshape(-1)
        extras = [] if g < K_GROUPS - 1 else [dense] + partials
        run = _make_sc_call(nf, len(extras))
        out = run(idx_g, *planes_g, *extras)
        if g < K_GROUPS - 1:
            partials.append(out)
        else:
            final = out
    return final.reshape(B, 1)


# speedup vs baseline: 1.2721x; 1.0083x over previous
import functools

import jax
import jax.numpy as jnp
from jax import lax
from jax.experimental import pallas as pl
from jax.experimental.pallas import tpu as pltpu
from jax.experimental.pallas import tpu_sc as plsc

B = 16384
N_SPARSE = 26
N_DENSE = 13
VOCAB = 100000
LANES = 16
K_GROUPS = 2

_info = plsc.get_sparse_core_info()
NC, NS = _info.num_cores, _info.num_subcores
NW = NC * NS
RPW = B // NW
CHUNKS = RPW // LANES


def _sc_group_body(n_fields, n_extra, f0, *refs):
    idx_hbm = refs[0]
    planes = refs[1:1 + n_fields]
    extras = refs[1 + n_fields:1 + n_fields + n_extra]
    out_hbm = refs[1 + n_fields + n_extra]
    idx_v, gat_v, ext_v, out_v, sem = refs[2 + n_fields + n_extra:]

    wid = lax.axis_index("s") * NC + lax.axis_index("c")
    base = wid * RPW
    seg = n_fields * RPW

    pltpu.sync_copy(
        idx_hbm.at[pl.ds(wid * N_SPARSE * RPW + f0 * RPW, seg)], idx_v)
    copies = [
        pltpu.async_copy(
            planes[j].at[idx_v.at[pl.ds(j * RPW, RPW)]],
            gat_v.at[pl.ds(j * RPW, RPW)], sem)
        for j in range(n_fields)
    ]
    for e in range(n_extra):
        pltpu.sync_copy(extras[e].at[pl.ds(base, RPW)],
                        ext_v.at[pl.ds(e * RPW, RPW)])
    for c in copies:
        c.wait()

    def reduce(c, carry):
        o = c * LANES
        acc = gat_v[pl.ds(o, LANES)]
        for j in range(1, n_fields):
            acc = acc + gat_v[pl.ds(j * RPW + o, LANES)]
        for e in range(n_extra):
            acc = acc + ext_v[pl.ds(e * RPW + o, LANES)]
        out_v[pl.ds(o, LANES)] = acc
        return carry

    lax.fori_loop(0, CHUNKS, reduce, 0)
    pltpu.sync_copy(out_v, out_hbm.at[pl.ds(base, RPW)])


def _make_sc_call(n_fields, n_extra, f0):
    mesh = plsc.VectorSubcoreMesh(core_axis_name="c", subcore_axis_name="s")
    return pl.kernel(
        functools.partial(_sc_group_body, n_fields, n_extra, f0),
        mesh=mesh,
        out_type=jax.ShapeDtypeStruct((B,), jnp.float32),
        scratch_types=[
            pltpu.VMEM((n_fields * RPW,), jnp.int32),
            pltpu.VMEM((n_fields * RPW,), jnp.float32),
            pltpu.VMEM((max(n_extra, 1) * RPW,), jnp.float32),
            pltpu.VMEM((RPW,), jnp.float32),
            pltpu.SemaphoreType.DMA,
        ],
        compiler_params=pltpu.CompilerParams(needs_layout_passes=False),
    )


def _dense_body(xdt_ref, w_ref, out_ref):
    out_ref[:] = jnp.sum(xdt_ref[:] * w_ref[:], axis=0)


def _dense_matvec(X_dense, weight):
    return pl.pallas_call(
        _dense_body,
        out_shape=jax.ShapeDtypeStruct((B,), jnp.float32),
        in_specs=[
            pl.BlockSpec((N_DENSE, B), lambda: (0, 0)),
            pl.BlockSpec((N_DENSE, 1), lambda: (0, 0)),
        ],
        out_specs=pl.BlockSpec((B,), lambda: (0,)),
    )(X_dense.T, weight)


@jax.jit
def kernel(X_sparse, X_dense, tables, weight):
    gsz = [16, 10]
    starts = [0, 16]

    idx_all = (X_sparse.reshape(NW, RPW, N_SPARSE)
               .transpose(0, 2, 1).reshape(-1))
    dense = _dense_matvec(X_dense, weight)

    partials = []
    for g in range(K_GROUPS):
        f0, nf = starts[g], gsz[g]
        planes_g = [tables[f, :, 0] for f in range(f0, f0 + nf)]
        extras = [] if g < K_GROUPS - 1 else [dense] + partials
        run = _make_sc_call(nf, len(extras), f0)
        out = run(idx_all, *planes_g, *extras)
        if g < K_GROUPS - 1:
            partials.append(out)
        else:
            final = out
    return final.reshape(B, 1)


# speedup vs baseline: 1.3367x; 1.0508x over previous
import functools

import jax
import jax.numpy as jnp
from jax import lax
from jax.experimental import pallas as pl
from jax.experimental.pallas import tpu as pltpu
from jax.experimental.pallas import tpu_sc as plsc

B = 16384
N_SPARSE = 26
N_DENSE = 13
VOCAB = 100000
LANES = 16
K_GROUPS = 3

_info = plsc.get_sparse_core_info()
NC, NS = _info.num_cores, _info.num_subcores
NW = NC * NS
RPW = B // NW
CHUNKS = RPW // LANES


def _sc_group_body(n_fields, n_extra, f0, *refs):
    idx_hbm = refs[0]
    planes = refs[1:1 + n_fields]
    extras = refs[1 + n_fields:1 + n_fields + n_extra]
    out_hbm = refs[1 + n_fields + n_extra]
    idx_v, gat_v, ext_v, out_v, sem = refs[2 + n_fields + n_extra:]

    wid = lax.axis_index("s") * NC + lax.axis_index("c")
    base = wid * RPW
    seg = n_fields * RPW

    pltpu.sync_copy(
        idx_hbm.at[pl.ds(wid * N_SPARSE * RPW + f0 * RPW, seg)], idx_v)
    copies = [
        pltpu.async_copy(
            planes[j].at[idx_v.at[pl.ds(j * RPW, RPW)]],
            gat_v.at[pl.ds(j * RPW, RPW)], sem)
        for j in range(n_fields)
    ]
    for e in range(n_extra):
        pltpu.sync_copy(extras[e].at[pl.ds(base, RPW)],
                        ext_v.at[pl.ds(e * RPW, RPW)])
    for c in copies:
        c.wait()

    def reduce(c, carry):
        o = c * LANES
        acc = gat_v[pl.ds(o, LANES)]
        for j in range(1, n_fields):
            acc = acc + gat_v[pl.ds(j * RPW + o, LANES)]
        for e in range(n_extra):
            acc = acc + ext_v[pl.ds(e * RPW + o, LANES)]
        out_v[pl.ds(o, LANES)] = acc
        return carry

    lax.fori_loop(0, CHUNKS, reduce, 0)
    pltpu.sync_copy(out_v, out_hbm.at[pl.ds(base, RPW)])


def _make_sc_call(n_fields, n_extra, f0):
    mesh = plsc.VectorSubcoreMesh(core_axis_name="c", subcore_axis_name="s")
    return pl.kernel(
        functools.partial(_sc_group_body, n_fields, n_extra, f0),
        mesh=mesh,
        out_type=jax.ShapeDtypeStruct((B,), jnp.float32),
        scratch_types=[
            pltpu.VMEM((n_fields * RPW,), jnp.int32),
            pltpu.VMEM((n_fields * RPW,), jnp.float32),
            pltpu.VMEM((max(n_extra, 1) * RPW,), jnp.float32),
            pltpu.VMEM((RPW,), jnp.float32),
            pltpu.SemaphoreType.DMA,
        ],
        compiler_params=pltpu.CompilerParams(needs_layout_passes=False),
    )


def _dense_body(xdt_ref, w_ref, out_ref):
    out_ref[:] = jnp.sum(xdt_ref[:] * w_ref[:], axis=0)


def _dense_matvec(X_dense, weight):
    return pl.pallas_call(
        _dense_body,
        out_shape=jax.ShapeDtypeStruct((B,), jnp.float32),
        in_specs=[
            pl.BlockSpec((N_DENSE, B), lambda: (0, 0)),
            pl.BlockSpec((N_DENSE, 1), lambda: (0, 0)),
        ],
        out_specs=pl.BlockSpec((B,), lambda: (0,)),
    )(X_dense.T, weight)


@jax.jit
def kernel(X_sparse, X_dense, tables, weight):
    gsz = [11, 9, 6]
    starts = [0, 11, 20]

    idx_all = (X_sparse.reshape(NW, RPW, N_SPARSE)
               .transpose(0, 2, 1).reshape(-1))
    dense = _dense_matvec(X_dense, weight)

    partials = []
    tab_src = tables
    for g in range(K_GROUPS):
        f0, nf = starts[g], gsz[g]
        planes_g = [tab_src[f, :, 0] for f in range(f0, f0 + nf)]
        tab_src = lax.optimization_barrier(tab_src)
        extras = [] if g < K_GROUPS - 1 else [dense] + partials
        run = _make_sc_call(nf, len(extras), f0)
        out = run(idx_all, *planes_g, *extras)
        if g < K_GROUPS - 1:
            partials.append(out)
        else:
            final = out
    return final.reshape(B, 1)
